# baseline (speedup 1.0000x reference)
import functools

import numpy as np
import jax
import jax.numpy as jnp
from jax.experimental import pallas as pl

D = 1024
H = 8
DH = 128
NH = 8
S = 52
V = 24
B = 128
BB = 8
RB = BB * S
NBLK = B // BB
NBH = S // 2

HI = jax.lax.Precision.HIGHEST
DE = jax.lax.Precision.DEFAULT


def _make_pe():
    pos = np.arange(S)[:, None].astype(np.float64)
    i = np.arange(D)[None, :]
    angle = pos / np.power(10000.0, (2 * (i // 2)) / float(D))
    pe = np.zeros((S, D))
    pe[:, 0::2] = np.sin(angle[:, 0::2])
    pe[:, 1::2] = np.cos(angle[:, 1::2])
    return jnp.asarray(pe, dtype=jnp.float32)


_PE = _make_pe()


def _dot(a, b, prec):
    return jax.lax.dot_general(a, b, (((1,), (0,)), ((), ())),
                               preferred_element_type=jnp.float32,
                               precision=prec)


def _dott(a, b, prec):
    return jax.lax.dot_general(a, b, (((1,), (1,)), ((), ())),
                               preferred_element_type=jnp.float32,
                               precision=prec)


def _ln(h, g, b):
    mu = jnp.mean(h, axis=1, keepdims=True)
    var = jnp.mean((h - mu) ** 2, axis=1, keepdims=True)
    return (h - mu) / jnp.sqrt(var + 1e-5) * g + b


def _attn_math(hin, xc, xl, g_ref, b_ref, wqk_ref, wv_ref, wo_ref, bo_ref,
               rot_ref, out_ref):
    mk_col = (xc != 0).astype(jnp.float32)
    mk_row = (xl != 0)
    hn = _ln(hin, g_ref[0], b_ref[0])

    ri = jax.lax.broadcasted_iota(jnp.int32, (RB, RB), 0)
    ci = jax.lax.broadcasted_iota(jnp.int32, (RB, RB), 1)
    bdiag = (ri // S) == (ci // S)
    eye = ri == ci
    keymask = jnp.broadcast_to(mk_row, (RB, RB))

    outs = []
    for h in range(H):
        qk = _dot(hn, wqk_ref[:, h * DH:(h + 1) * DH], DE)
        v = _dot(hn, wv_ref[:, h * DH:(h + 1) * DH], DE)
        rotd = _dot(qk, rot_ref[:], DE)
        ohs = []
        io = jax.lax.broadcasted_iota(jnp.int32, (RB, S), 1)
        for n in range(NH):
            rn = rotd[:, n * NBH:(n + 1) * NBH]
            full = jnp.concatenate([rn, -rn], axis=1)
            mx = jnp.max(full, axis=1, keepdims=True)
            idx = jnp.min(jnp.where(full >= mx, io, 2 * S),
                          axis=1, keepdims=True)
            ohs.append((io == idx).astype(jnp.float32))
        oh = jnp.concatenate(ohs, axis=1)
        cnt = _dott(oh, oh, DE)
        same = cnt > 0.5

        nrm = jnp.sqrt(jnp.sum(qk * qk, axis=1, keepdims=True))
        kk = qk / (nrm + 1e-8)
        s = _dott(qk, kk, DE) / np.float32(np.sqrt(float(DH)))
        s = jnp.where(same & bdiag & keymask, s, -1e9)
        s = jnp.where(eye, np.float32(-1e5), s)
        m = jnp.max(s, axis=1, keepdims=True)
        e = jnp.exp(s - m)
        p = e / jnp.sum(e, axis=1, keepdims=True)
        outs.append(_dot(p, v, DE))
    att = jnp.concatenate(outs, axis=1)
    o = _dot(att, wo_ref[:], DE) + bo_ref[0]
    out_ref[:] = hin + mk_col * o


def _attn_embed_kernel(xl_ref, xc_ref, emb_ref, pe_ref, g_ref, b_ref,
                       wqk_ref, wv_ref, wo_ref, bo_ref, rot_ref, out_ref):
    xc = xc_ref[0]
    onehot = (xc == jax.lax.broadcasted_iota(jnp.int32, (RB, V), 1))
    hin = _dot(onehot.astype(jnp.float32), emb_ref[:], HI) + pe_ref[:]
    _attn_math(hin, xc, xl_ref[0], g_ref, b_ref, wqk_ref, wv_ref, wo_ref,
               bo_ref, rot_ref, out_ref)


def _attn_kernel(xl_ref, xc_ref, hin_ref, g_ref, b_ref,
                 wqk_ref, wv_ref, wo_ref, bo_ref, rot_ref, out_ref):
    _attn_math(hin_ref[:], xc_ref[0], xl_ref[0], g_ref, b_ref, wqk_ref,
               wv_ref, wo_ref, bo_ref, rot_ref, out_ref)


def _ff_kernel(hin_ref, g_ref, b_ref, w1_ref, b1_ref, w2_ref, b2_ref,
               out_ref):
    hin = hin_ref[:]
    hn = _ln(hin, g_ref[0], b_ref[0])
    a = _dot(hn, w1_ref[:], DE) + b1_ref[0]
    gg = jax.nn.gelu(a)
    out_ref[:] = hin + _dot(gg, w2_ref[:], DE) + b2_ref[0]


def _whole(arr):
    nd = arr.ndim
    return pl.BlockSpec(arr.shape, lambda i, _nd=nd: (0,) * _nd)


def kernel(x, params):
    x = x.astype(jnp.int32)
    layers = params['layers']
    emb = params['token_emb']
    pet = jnp.tile(_PE, (BB, 1))
    rot = jax.random.normal(jax.random.key(42), (DH, NH, NBH),
                            dtype=jnp.float32).reshape(DH, NH * NBH)

    xf = x.reshape(-1)
    xl = xf.reshape(NBLK, 1, RB)
    xc = xf.reshape(NBLK, RB, 1)

    def row2(a):
        return a.reshape(1, -1)

    def attn_call(l, hin):
        g, b = row2(l['ln1_g']), row2(l['ln1_b'])
        bo = row2(l['bo'])
        common = [l['wqk'], l['wv'], l['wo'], bo, rot]
        io_spec = pl.BlockSpec((RB, D), lambda i: (i, 0))
        xl_spec = pl.BlockSpec((1, 1, RB), lambda i: (i, 0, 0))
        xc_spec = pl.BlockSpec((1, RB, 1), lambda i: (i, 0, 0))
        if hin is None:
            args = [xl, xc, emb, pet, g, b] + common
            kfn = _attn_embed_kernel
            in_specs = [xl_spec, xc_spec] + [_whole(a) for a in args[2:]]
        else:
            args = [xl, xc, hin, g, b] + common
            kfn = _attn_kernel
            in_specs = [xl_spec, xc_spec, io_spec] + [_whole(a)
                                                      for a in args[3:]]
        return pl.pallas_call(
            kfn,
            grid=(NBLK,),
            in_specs=in_specs,
            out_specs=io_spec,
            out_shape=jax.ShapeDtypeStruct((B * S, D), jnp.float32),
        )(*args)

    def ff_call(l, hin):
        g, b = row2(l['ln2_g']), row2(l['ln2_b'])
        b1, b2 = row2(l['b1']), row2(l['b2'])
        rows = hin.shape[0]
        blk = 208 if rows % 208 == 0 else rows
        grid = rows // blk
        io_spec = pl.BlockSpec((blk, D), lambda i: (i, 0))
        args = [hin, g, b, l['w1'], b1, l['w2'], b2]
        return pl.pallas_call(
            _ff_kernel,
            grid=(grid,),
            in_specs=[io_spec] + [_whole(a) for a in args[1:]],
            out_specs=io_spec,
            out_shape=jax.ShapeDtypeStruct((rows, D), jnp.float32),
        )(*args)

    h = attn_call(layers[0], None)
    h = ff_call(layers[0], h)
    h = attn_call(layers[1], h)
    h0 = h.reshape(B, S, D)[:, 0, :]
    out = ff_call(layers[1], h0)
    return out


# speedup vs baseline: 2.4497x; 2.4497x over previous
import functools

import numpy as np
import jax
import jax.numpy as jnp
from jax.experimental import pallas as pl

D = 1024
H = 8
DH = 128
NH = 8
S = 52
V = 24
B = 128
BB = 8
RB = BB * S
NBLK = B // BB
NBH = S // 2

HI = jax.lax.Precision.HIGHEST
DE = jax.lax.Precision.DEFAULT


def _make_pe():
    pos = np.arange(S)[:, None].astype(np.float64)
    i = np.arange(D)[None, :]
    angle = pos / np.power(10000.0, (2 * (i // 2)) / float(D))
    pe = np.zeros((S, D))
    pe[:, 0::2] = np.sin(angle[:, 0::2])
    pe[:, 1::2] = np.cos(angle[:, 1::2])
    return pe.astype(np.float32)


_PE_NP = _make_pe()


def _dot(a, b, prec):
    return jax.lax.dot_general(a, b, (((1,), (0,)), ((), ())),
                               preferred_element_type=jnp.float32,
                               precision=prec)


def _dott(a, b, prec):
    return jax.lax.dot_general(a, b, (((1,), (1,)), ((), ())),
                               preferred_element_type=jnp.float32,
                               precision=prec)


def _ln(h, g, b):
    mu = jnp.mean(h, axis=1, keepdims=True)
    var = jnp.mean((h - mu) ** 2, axis=1, keepdims=True)
    return (h - mu) / jnp.sqrt(var + 1e-5) * g + b


def _attn_math(hin, xc, xl, g_ref, b_ref, wqk_ref, wv_ref, wo_ref, bo_ref,
               rot_ref, out_ref):
    mk_col = (xc != 0).astype(jnp.float32)
    mk_row = (xl != 0)
    hn = _ln(hin, g_ref[0], b_ref[0])

    ri = jax.lax.broadcasted_iota(jnp.int32, (RB, RB), 0)
    ci = jax.lax.broadcasted_iota(jnp.int32, (RB, RB), 1)
    bdiag = (ri // S) == (ci // S)
    eye = ri == ci
    keymask = jnp.broadcast_to(mk_row, (RB, RB))

    qk_all = _dot(hn, wqk_ref[:], DE)
    v_all = _dot(hn, wv_ref[:], DE)
    qk_t = jnp.transpose(qk_all)

    outs = []
    for h in range(H):
        qk = qk_all[:, h * DH:(h + 1) * DH]
        rott = _dot(rot_ref[:], qk_t[h * DH:(h + 1) * DH, :], DE)
        r3 = rott.reshape(NH, 64, RB)
        mx = jnp.max(r3, axis=1, keepdims=True)
        io3 = jax.lax.broadcasted_iota(jnp.int32, (NH, 64, RB), 1)
        idx = jnp.min(jnp.where(r3 >= mx, io3, 64), axis=1, keepdims=True)
        oh_t = (io3 == idx).astype(jnp.float32).reshape(NH * 64, RB)
        cnt = jax.lax.dot_general(oh_t, oh_t, (((0,), (0,)), ((), ())),
                                  preferred_element_type=jnp.float32,
                                  precision=DE)
        same = cnt > 0.5

        nrm = jnp.sqrt(jnp.sum(qk * qk, axis=1, keepdims=True))
        kk = qk / (nrm + 1e-8)
        s = _dott(qk, kk, DE) / np.float32(np.sqrt(float(DH)))
        s = jnp.where(same & bdiag & keymask, s, -1e9)
        s = jnp.where(eye, np.float32(-1e5), s)
        m = jnp.max(s, axis=1, keepdims=True)
        e = jnp.exp(s - m)
        p = e / jnp.sum(e, axis=1, keepdims=True)
        outs.append(_dot(p, v_all[:, h * DH:(h + 1) * DH], DE))
    att = jnp.concatenate(outs, axis=1)
    o = _dot(att, wo_ref[:], DE) + bo_ref[0]
    out_ref[:] = hin + mk_col * o


def _attn_embed_kernel(xl_ref, xc_ref, emb_ref, pe_ref, g_ref, b_ref,
                       wqk_ref, wv_ref, wo_ref, bo_ref, rot_ref, out_ref):
    xc = xc_ref[0]
    onehot = (xc == jax.lax.broadcasted_iota(jnp.int32, (RB, V), 1))
    hin = _dot(onehot.astype(jnp.float32), emb_ref[:], HI) + pe_ref[:]
    _attn_math(hin, xc, xl_ref[0], g_ref, b_ref, wqk_ref, wv_ref, wo_ref,
               bo_ref, rot_ref, out_ref)


def _attn_kernel(xl_ref, xc_ref, hin_ref, g_ref, b_ref,
                 wqk_ref, wv_ref, wo_ref, bo_ref, rot_ref, out_ref):
    _attn_math(hin_ref[:], xc_ref[0], xl_ref[0], g_ref, b_ref, wqk_ref,
               wv_ref, wo_ref, bo_ref, rot_ref, out_ref)


def _ff_kernel(hin_ref, g_ref, b_ref, w1_ref, b1_ref, w2_ref, b2_ref,
               out_ref):
    hin = hin_ref[:]
    hn = _ln(hin, g_ref[0], b_ref[0])
    a = _dot(hn, w1_ref[:], DE) + b1_ref[0]
    gg = jax.nn.gelu(a)
    out_ref[:] = hin + _dot(gg, w2_ref[:], DE) + b2_ref[0]


def _whole(arr):
    nd = arr.ndim
    return pl.BlockSpec(arr.shape, lambda i, _nd=nd: (0,) * _nd)


def kernel(x, params):
    x = x.astype(jnp.int32)
    layers = params['layers']
    emb = params['token_emb']
    pet = jnp.asarray(np.tile(_PE_NP, (BB, 1)))
    rot3 = jax.random.normal(jax.random.key(42), (DH, NH, NBH),
                             dtype=jnp.float32)
    rt = jnp.transpose(rot3, (1, 2, 0))
    rt = jnp.concatenate([rt, -rt], axis=1)
    rt = jnp.pad(rt, ((0, 0), (0, 64 - S), (0, 0)))
    rot = rt.reshape(NH * 64, DH)

    xf = x.reshape(-1)
    xl = xf.reshape(NBLK, 1, RB)
    xc = xf.reshape(NBLK, RB, 1)

    def row2(a):
        return a.reshape(1, -1)

    def attn_call(l, hin):
        g, b = row2(l['ln1_g']), row2(l['ln1_b'])
        bo = row2(l['bo'])
        common = [l['wqk'], l['wv'], l['wo'], bo, rot]
        io_spec = pl.BlockSpec((RB, D), lambda i: (i, 0))
        xl_spec = pl.BlockSpec((1, 1, RB), lambda i: (i, 0, 0))
        xc_spec = pl.BlockSpec((1, RB, 1), lambda i: (i, 0, 0))
        if hin is None:
            args = [xl, xc, emb, pet, g, b] + common
            kfn = _attn_embed_kernel
            in_specs = [xl_spec, xc_spec] + [_whole(a) for a in args[2:]]
        else:
            args = [xl, xc, hin, g, b] + common
            kfn = _attn_kernel
            in_specs = [xl_spec, xc_spec, io_spec] + [_whole(a)
                                                      for a in args[3:]]
        return pl.pallas_call(
            kfn,
            grid=(NBLK,),
            in_specs=in_specs,
            out_specs=io_spec,
            out_shape=jax.ShapeDtypeStruct((B * S, D), jnp.float32),
        )(*args)

    def ff_call(l, hin):
        g, b = row2(l['ln2_g']), row2(l['ln2_b'])
        b1, b2 = row2(l['b1']), row2(l['b2'])
        rows = hin.shape[0]
        blk = 208 if rows % 208 == 0 else rows
        grid = rows // blk
        io_spec = pl.BlockSpec((blk, D), lambda i: (i, 0))
        args = [hin, g, b, l['w1'], b1, l['w2'], b2]
        return pl.pallas_call(
            _ff_kernel,
            grid=(grid,),
            in_specs=[io_spec] + [_whole(a) for a in args[1:]],
            out_specs=io_spec,
            out_shape=jax.ShapeDtypeStruct((rows, D), jnp.float32),
        )(*args)

    h = attn_call(layers[0], None)
    h = ff_call(layers[0], h)
    h = attn_call(layers[1], h)
    h0 = h.reshape(B, S, D)[:, 0, :]
    out = ff_call(layers[1], h0)
    return out
